# baseline (speedup 1.0000x reference)
import jax
import jax.numpy as jnp
from jax import lax
from jax.experimental import pallas as pl
from jax.experimental.pallas import tpu as pltpu
from jax.experimental.pallas import tpu_sc as plsc

_DMODEL = 1024
_B = 4
_T = 2048

_NC = 2
_NS = 16
_NW = _NC * _NS
_TPW = _T // _NW
_CHUNK = 16
_NCH = _TPW // _CHUNK
_NBUF = 4
_PHALF = _TPW // 2
_LANES = 16
_VPR = _DMODEL // _LANES
_NIT = _NCH * _B


def _emb_body(x_hbm, tok_hbm, pos_hbm, out_hbm,
              idx_v, pos_v, rows_v, gsems, wsems, psem, isem):
    wid = lax.axis_index("s") * _NC + lax.axis_index("c")
    tb = wid * _TPW

    icps = [pltpu.async_copy(x_hbm.at[pl.ds(b * _T + tb, _TPW)],
                             idx_v.at[b], isem) for b in range(_B)]
    pos_cp = pltpu.async_copy(pos_hbm.at[pl.ds(tb, _PHALF)], pos_v, psem)
    for cp in icps:
        cp.wait()

    def chunk_cb(i):
        return i // _B, i % _B

    def start_gather(i, p):
        c, b = chunk_cb(i)
        return pltpu.async_copy(
            tok_hbm.at[idx_v.at[b, pl.ds(c * _CHUNK, _CHUNK)]],
            rows_v[p], gsems[p])

    def start_write(i, p):
        c, b = chunk_cb(i)
        base = b * _T + tb + c * _CHUNK
        return pltpu.async_copy(rows_v[p], out_hbm.at[pl.ds(base, _CHUNK)],
                                wsems[p])

    g = [None] * _NBUF
    w = [None] * _NBUF
    for j in range(_NBUF - 1):
        g[j] = start_gather(j, j)
    pos_cp.wait()

    mid = _NIT // 2
    for i in range(_NIT):
        p = i % _NBUF
        nxt = i + _NBUF - 1
        if nxt < _NIT:
            q = nxt % _NBUF
            if w[q] is not None:
                w[q].wait()
                w[q] = None
            g[q] = start_gather(nxt, q)
        if i == mid:
            pos_cp = pltpu.async_copy(pos_hbm.at[pl.ds(tb + _PHALF, _PHALF)],
                                      pos_v, psem)
        g[p].wait()
        if i == mid:
            pos_cp.wait()

        c, _ = chunk_cb(i)
        prow = (c % (_PHALF // _CHUNK)) * _CHUNK
        buf = rows_v[p]

        def add_col(j, carry, prow=prow, buf=buf):
            col = pl.ds(j * _LANES, _LANES)
            for r in range(_CHUNK):
                buf[r, col] = buf[r, col] + pos_v[prow + r, col]
            return carry

        lax.fori_loop(0, _VPR, add_col, 0)
        w[p] = start_write(i, p)

    for p in range(_NBUF):
        if w[p] is not None:
            w[p].wait()


@jax.jit
def kernel(x, tokemb, posemb):
    b, t = x.shape
    mesh = plsc.VectorSubcoreMesh(core_axis_name="c", subcore_axis_name="s")
    out = pl.kernel(
        _emb_body,
        out_type=jax.ShapeDtypeStruct((b * t, _DMODEL), jnp.float32),
        mesh=mesh,
        scratch_types=[
            pltpu.VMEM((_B, _TPW), jnp.int32),
            pltpu.VMEM((_PHALF, _DMODEL), jnp.float32),
            [pltpu.VMEM((_CHUNK, _DMODEL), jnp.float32)] * _NBUF,
            [pltpu.SemaphoreType.DMA] * _NBUF,
            [pltpu.SemaphoreType.DMA] * _NBUF,
            pltpu.SemaphoreType.DMA,
            pltpu.SemaphoreType.DMA,
        ],
    )(x.reshape(b * t).astype(jnp.int32), tokemb, posemb)
    return out.reshape(b, t, _DMODEL)
